# baseline (speedup 1.0000x reference)
import functools

import jax
import jax.numpy as jnp
from jax import lax
from jax.experimental import pallas as pl
from jax.experimental.pallas import tpu as pltpu
from jax.experimental.pallas import tpu_sc as plsc

N = 50000
NS = 12500
NCAND = 12544
CIN = 256
COUT = 128

QB = 400
QBF = 1000

SC_CORES = 2
SC_SUBCORES = 16
SC_WORKERS = SC_CORES * SC_SUBCORES
GCHUNK = 128
BSC = 155648


def _mlp_sub_body(xs_ref, w_ref, b_ref, o_ref):
    acc = jnp.dot(xs_ref[...], w_ref[...],
                  preferred_element_type=jnp.float32,
                  precision=lax.Precision.HIGHEST)
    o_ref[...] = jnp.maximum(acc + b_ref[...], 0.0)


NGRP = 8
GW = NCAND // NGRP


def _knn_body(pos_ref, psub_ref, idx_ref, w_ref):
    q = pos_ref[...]
    p = psub_ref[...]
    qx = q[:, 0:1][:, :, None]
    qy = q[:, 1:2][:, :, None]
    qz = q[:, 2:3][:, :, None]
    dx = qx - p[0][None]
    dy = qy - p[1][None]
    dz = qz - p[2][None]
    d2r = dx * dx + dy * dy + dz * dz
    big = jnp.float32(1e30)

    gm = jnp.min(d2r, axis=2)
    giota = lax.broadcasted_iota(jnp.int32, (QB, NGRP), 1).astype(jnp.float32)
    gsel = []
    for r in range(3):
        m = jnp.min(gm, axis=1, keepdims=True)
        g = jnp.min(jnp.where(gm == m, giota, big), axis=1, keepdims=True)
        gsel.append(g)
        if r < 2:
            gm = jnp.where(giota == g, big, gm)
    g1, g2, g3 = gsel
    s1 = jnp.minimum(jnp.minimum(g1, g2), g3)
    s3 = jnp.maximum(jnp.maximum(g1, g2), g3)
    s2 = g1 + g2 + g3 - s1 - s3
    gcat = jnp.concatenate([s1, s2, s3], axis=1)

    gidx = gcat.astype(jnp.int32)[:, :, None]
    gsub = jnp.take_along_axis(d2r, jnp.broadcast_to(gidx, (QB, 3, GW)),
                               axis=1)
    d2s = gsub.reshape(QB, 3 * GW)
    liota = lax.broadcasted_iota(jnp.int32, (QB, 3 * GW), 1).astype(
        jnp.float32)
    ms, idxs = [], []
    for r in range(3):
        m = jnp.min(d2s, axis=1, keepdims=True)
        a = jnp.min(jnp.where(d2s == m, liota, big), axis=1,
                    keepdims=True)
        k = jnp.floor(a * (1.0 / GW))
        pos = a - k * GW
        gv = jnp.take_along_axis(gcat, k.astype(jnp.int32), axis=1)
        ms.append(m)
        idxs.append(gv * GW + pos)
        if r < 2:
            d2s = jnp.where(liota == a, big, d2s)
    d_sel = jnp.concatenate(ms, axis=1)
    idx_ref[...] = jnp.concatenate(idxs, axis=1).astype(jnp.int32)
    wk = 1.0 / jnp.maximum(d_sel, 1e-16)
    w_ref[...] = wk / jnp.sum(wk, axis=1, keepdims=True)


def _combine_body(x_ref, w_mat_ref, b_ref, w_ref, g0_ref, g1_ref, g2_ref,
                  o_ref):
    h = jnp.dot(x_ref[...], w_mat_ref[...],
                preferred_element_type=jnp.float32,
                precision=lax.Precision.HIGHEST)
    h = jnp.maximum(h + b_ref[...], 0.0)
    w = w_ref[...]
    o_ref[...] = (h + w[:, 0:1] * g0_ref[...]
                  + w[:, 1:2] * g1_ref[...]
                  + w[:, 2:3] * g2_ref[...])


def _sc_gather(table, idx_flat, bsc):
    mesh = plsc.VectorSubcoreMesh(core_axis_name="c", subcore_axis_name="s")
    b_per_w = bsc // SC_WORKERS
    nchunk = b_per_w // GCHUNK
    assert nchunk % 2 == 0

    @functools.partial(
        pl.kernel,
        out_type=jax.ShapeDtypeStruct((bsc, COUT), jnp.float32),
        mesh=mesh,
        scratch_types=[
            pltpu.VMEM((GCHUNK,), jnp.int32),
            pltpu.VMEM((GCHUNK,), jnp.int32),
            pltpu.VMEM((GCHUNK, COUT), jnp.float32),
            pltpu.VMEM((GCHUNK, COUT), jnp.float32),
            pltpu.SemaphoreType.DMA,
            pltpu.SemaphoreType.DMA,
        ],
    )
    def gather_kernel(table_hbm, idx_hbm, out_hbm, idx_v0, idx_v1,
                      rows_v0, rows_v1, sem0, sem1):
        wid = lax.axis_index("s") * SC_CORES + lax.axis_index("c")
        base = wid * b_per_w

        @pl.loop(0, nchunk, step=2)
        def _(c):
            off0 = base + c * GCHUNK
            off1 = off0 + GCHUNK
            pltpu.sync_copy(idx_hbm.at[pl.ds(off0, GCHUNK)], idx_v0)
            h0 = pltpu.async_copy(table_hbm.at[idx_v0], rows_v0, sem0)
            pltpu.sync_copy(idx_hbm.at[pl.ds(off1, GCHUNK)], idx_v1)
            h1 = pltpu.async_copy(table_hbm.at[idx_v1], rows_v1, sem1)
            h0.wait()
            pltpu.sync_copy(rows_v0, out_hbm.at[pl.ds(off0, GCHUNK)])
            h1.wait()
            pltpu.sync_copy(rows_v1, out_hbm.at[pl.ds(off1, GCHUNK)])

    return gather_kernel(table, idx_flat)


def kernel(x, x_sub, pos, pos_sub, W_sub, b_sub, W, b):
    qs = 1568
    h_sub = pl.pallas_call(
        _mlp_sub_body,
        grid=(NCAND // qs,),
        in_specs=[
            pl.BlockSpec((qs, CIN), lambda i: (i, 0)),
            pl.BlockSpec((CIN, COUT), lambda i: (0, 0)),
            pl.BlockSpec((1, COUT), lambda i: (0, 0)),
        ],
        out_specs=pl.BlockSpec((qs, COUT), lambda i: (i, 0)),
        out_shape=jax.ShapeDtypeStruct((NS, COUT), jnp.float32),
    )(x_sub, W_sub, b_sub.reshape(1, COUT))

    psub_t = jnp.pad(pos_sub, ((0, NCAND - NS), (0, 0)),
                     constant_values=100.0).T.reshape(3, NGRP, GW)
    idx, w = pl.pallas_call(
        _knn_body,
        grid=(N // QB,),
        in_specs=[
            pl.BlockSpec((QB, 3), lambda i: (i, 0)),
            pl.BlockSpec((3, NGRP, GW), lambda i: (0, 0, 0)),
        ],
        out_specs=[
            pl.BlockSpec((QB, 3), lambda i: (i, 0)),
            pl.BlockSpec((QB, 3), lambda i: (i, 0)),
        ],
        out_shape=[
            jax.ShapeDtypeStruct((N, 3), jnp.int32),
            jax.ShapeDtypeStruct((N, 3), jnp.float32),
        ],
    )(pos, psub_t)

    idx_flat = jnp.pad(idx.T.reshape(3 * N), (0, BSC - 3 * N))
    gathered = _sc_gather(h_sub, idx_flat, BSC)

    nqb = N // QBF
    gspec = lambda k: pl.BlockSpec((QBF, COUT),
                                   lambda i, k=k: (k * nqb + i, 0))
    out = pl.pallas_call(
        _combine_body,
        grid=(nqb,),
        in_specs=[
            pl.BlockSpec((QBF, COUT), lambda i: (i, 0)),
            pl.BlockSpec((COUT, COUT), lambda i: (0, 0)),
            pl.BlockSpec((1, COUT), lambda i: (0, 0)),
            pl.BlockSpec((QBF, 3), lambda i: (i, 0)),
            gspec(0),
            gspec(1),
            gspec(2),
        ],
        out_specs=pl.BlockSpec((QBF, COUT), lambda i: (i, 0)),
        out_shape=jax.ShapeDtypeStruct((N, COUT), jnp.float32),
    )(x, W, b.reshape(1, COUT), w, gathered, gathered, gathered)
    return out


# speedup vs baseline: 1.0969x; 1.0969x over previous
import functools

import jax
import jax.numpy as jnp
from jax import lax
from jax.experimental import pallas as pl
from jax.experimental.pallas import tpu as pltpu
from jax.experimental.pallas import tpu_sc as plsc

N = 50000
NS = 12500
NCAND = 12544
CIN = 256
COUT = 128

QB = 400
QBF = 1000

SC_CORES = 2
SC_SUBCORES = 16
SC_WORKERS = SC_CORES * SC_SUBCORES
GCHUNK = 128
BSC = 151552


def _mlp_sub_body(xs_ref, w_ref, b_ref, o_ref):
    acc = jnp.dot(xs_ref[...], w_ref[...],
                  preferred_element_type=jnp.float32,
                  precision=lax.Precision.HIGHEST)
    o_ref[...] = jnp.maximum(acc + b_ref[...], 0.0)


NGRP = 8
GW = NCAND // NGRP


def _knn_body(pos_ref, psub_ref, idx_ref, w_ref):
    q = pos_ref[...]
    p = psub_ref[...]
    qx = q[:, 0:1][:, :, None]
    qy = q[:, 1:2][:, :, None]
    qz = q[:, 2:3][:, :, None]
    dx = qx - p[0][None]
    dy = qy - p[1][None]
    dz = qz - p[2][None]
    d2r = dx * dx + dy * dy + dz * dz
    big = jnp.float32(1e30)

    gm = jnp.min(d2r, axis=2)
    giota = lax.broadcasted_iota(jnp.int32, (QB, NGRP), 1).astype(jnp.float32)
    gsel = []
    m1_global = None
    for r in range(3):
        m = jnp.min(gm, axis=1, keepdims=True)
        if r == 0:
            m1_global = m
        g = jnp.min(jnp.where(gm == m, giota, big), axis=1, keepdims=True)
        gsel.append(g)
        if r < 2:
            gm = jnp.where(giota == g, big, gm)
    g1, g2, g3 = gsel
    s1 = jnp.minimum(jnp.minimum(g1, g2), g3)
    s3 = jnp.maximum(jnp.maximum(g1, g2), g3)
    s2 = g1 + g2 + g3 - s1 - s3
    gcat = jnp.concatenate([s1, s2, s3], axis=1)

    gidx = gcat.astype(jnp.int32)[:, :, None]
    gsub = jnp.take_along_axis(d2r, jnp.broadcast_to(gidx, (QB, 3, GW)),
                               axis=1)
    d2s = gsub.reshape(QB, 3 * GW)
    liota = lax.broadcasted_iota(jnp.int32, (QB, 3 * GW), 1).astype(
        jnp.float32)
    ms, idxs = [], []
    for r in range(3):
        if r == 0:
            m = m1_global
        else:
            m = jnp.min(d2s, axis=1, keepdims=True)
        a = jnp.min(jnp.where(d2s == m, liota, big), axis=1,
                    keepdims=True)
        k = jnp.floor(a * (1.0 / GW))
        pos = a - k * GW
        gv = jnp.take_along_axis(gcat, k.astype(jnp.int32), axis=1)
        ms.append(m)
        idxs.append(gv * GW + pos)
        if r < 2:
            d2s = jnp.where(liota == a, big, d2s)
    d_sel = jnp.concatenate(ms, axis=1)
    idx_ref[...] = jnp.concatenate(idxs, axis=1).astype(jnp.int32)
    wk = 1.0 / jnp.maximum(d_sel, 1e-16)
    w_ref[...] = wk / jnp.sum(wk, axis=1, keepdims=True)


def _combine_body(x_ref, w_mat_ref, b_ref, w_ref, g0_ref, g1_ref, g2_ref,
                  o_ref):
    h = jnp.dot(x_ref[...], w_mat_ref[...],
                preferred_element_type=jnp.float32,
                precision=lax.Precision.HIGHEST)
    h = jnp.maximum(h + b_ref[...], 0.0)
    w = w_ref[...]
    o_ref[...] = (h + w[:, 0:1] * g0_ref[...]
                  + w[:, 1:2] * g1_ref[...]
                  + w[:, 2:3] * g2_ref[...])


def _sc_gather(table, idx_flat, bsc):
    mesh = plsc.VectorSubcoreMesh(core_axis_name="c", subcore_axis_name="s")
    b_per_w = bsc // SC_WORKERS
    nchunk = b_per_w // GCHUNK

    @functools.partial(
        pl.kernel,
        out_type=jax.ShapeDtypeStruct((bsc, COUT), jnp.float32),
        mesh=mesh,
        scratch_types=[
            pltpu.VMEM((GCHUNK,), jnp.int32),
            pltpu.VMEM((GCHUNK, COUT), jnp.float32),
            pltpu.SemaphoreType.DMA,
        ],
    )
    def gather_kernel(table_hbm, idx_hbm, out_hbm, idx_v, rows_v, sem):
        wid = lax.axis_index("s") * SC_CORES + lax.axis_index("c")
        base = wid * b_per_w

        @pl.loop(0, nchunk)
        def _(c):
            off = base + c * GCHUNK
            pltpu.sync_copy(idx_hbm.at[pl.ds(off, GCHUNK)], idx_v)
            pltpu.async_copy(table_hbm.at[idx_v], rows_v, sem).wait()
            pltpu.sync_copy(rows_v, out_hbm.at[pl.ds(off, GCHUNK)])

    return gather_kernel(table, idx_flat)


def kernel(x, x_sub, pos, pos_sub, W_sub, b_sub, W, b):
    qs = 1568
    h_sub = pl.pallas_call(
        _mlp_sub_body,
        grid=(NCAND // qs,),
        in_specs=[
            pl.BlockSpec((qs, CIN), lambda i: (i, 0)),
            pl.BlockSpec((CIN, COUT), lambda i: (0, 0)),
            pl.BlockSpec((1, COUT), lambda i: (0, 0)),
        ],
        out_specs=pl.BlockSpec((qs, COUT), lambda i: (i, 0)),
        out_shape=jax.ShapeDtypeStruct((NS, COUT), jnp.float32),
    )(x_sub, W_sub, b_sub.reshape(1, COUT))

    psub_t = jnp.pad(pos_sub, ((0, NCAND - NS), (0, 0)),
                     constant_values=100.0).T.reshape(3, NGRP, GW)
    idx, w = pl.pallas_call(
        _knn_body,
        grid=(N // QB,),
        in_specs=[
            pl.BlockSpec((QB, 3), lambda i: (i, 0)),
            pl.BlockSpec((3, NGRP, GW), lambda i: (0, 0, 0)),
        ],
        out_specs=[
            pl.BlockSpec((QB, 3), lambda i: (i, 0)),
            pl.BlockSpec((QB, 3), lambda i: (i, 0)),
        ],
        out_shape=[
            jax.ShapeDtypeStruct((N, 3), jnp.int32),
            jax.ShapeDtypeStruct((N, 3), jnp.float32),
        ],
    )(pos, psub_t)

    idx_flat = jnp.pad(idx.T.reshape(3 * N), (0, BSC - 3 * N))
    gathered = _sc_gather(h_sub, idx_flat, BSC)

    nqb = N // QBF
    gspec = lambda k: pl.BlockSpec((QBF, COUT),
                                   lambda i, k=k: (k * nqb + i, 0))
    out = pl.pallas_call(
        _combine_body,
        grid=(nqb,),
        in_specs=[
            pl.BlockSpec((QBF, COUT), lambda i: (i, 0)),
            pl.BlockSpec((COUT, COUT), lambda i: (0, 0)),
            pl.BlockSpec((1, COUT), lambda i: (0, 0)),
            pl.BlockSpec((QBF, 3), lambda i: (i, 0)),
            gspec(0),
            gspec(1),
            gspec(2),
        ],
        out_specs=pl.BlockSpec((QBF, COUT), lambda i: (i, 0)),
        out_shape=jax.ShapeDtypeStruct((N, COUT), jnp.float32),
    )(x, W, b.reshape(1, COUT), w, gathered, gathered, gathered)
    return out
